# baseline (speedup 1.0000x reference)
import dataclasses
import functools

import jax
import jax.numpy as jnp
import numpy as np
from jax import lax
from jax.experimental import pallas as pl
from jax.experimental.pallas import tpu as pltpu
from jax.experimental.pallas import tpu_sc as plsc

N = 10000
E = 320000
D = 128
ED = 16
B = 64

_DEG_HIST = np.array([0] * 32 + [10000], dtype=np.float64)
_bins_tmp = np.arange(len(_DEG_HIST))
AVG_LOG = float((np.log(_bins_tmp + 1.0) * _DEG_HIST).sum() / _DEG_HIST.sum())

_NC = 2
_NS = 16
_NW = _NC * _NS
_C = 128

_f32 = jnp.float32
_i32 = jnp.int32


def _ec_matmuls(ea, We1, Wp1, We2, Wp2, We3, Wp3):
  BM = 2000
  grid = (E // BM,)

  def body(a_ref, we1, wp1, we2, wp2, we3, wp3, o1, o2, o3):
    a = a_ref[...]
    for we, wp, o in ((we1, wp1, o1), (we2, wp2, o2), (we3, wp3, o3)):
      w = jnp.dot(we[...], wp[...], preferred_element_type=_f32)
      o[...] = jnp.dot(a, w, preferred_element_type=_f32)

  wspec = pl.BlockSpec((ED, D), lambda i: (0, 0))
  pspec = pl.BlockSpec((D, D), lambda i: (0, 0))
  return pl.pallas_call(
      body,
      grid=grid,
      in_specs=[pl.BlockSpec((BM, ED), lambda i: (i, 0)),
                wspec, pspec, wspec, pspec, wspec, pspec],
      out_specs=[pl.BlockSpec((BM, D), lambda i: (i, 0))] * 3,
      out_shape=[jax.ShapeDtypeStruct((E, D), _f32)] * 3,
  )(ea, We1, Wp1, We2, Wp2, We3, Wp3)


def _node_pre(h, Wab, WpC, be2d, bpre2d):
  BM = 1000
  grid = (N // BM,)

  def body(h_ref, wab, wpc, be_r, bp_r, o1, o2):
    c0 = jnp.dot(be_r[...], wpc[...], preferred_element_type=_f32) + bp_r[...]
    r = jnp.dot(h_ref[...], wab[...], preferred_element_type=_f32)
    o1[...] = r[:, 0:D] + c0
    o2[...] = r[:, D:2 * D]

  return pl.pallas_call(
      body,
      grid=grid,
      in_specs=[pl.BlockSpec((BM, D), lambda i: (i, 0)),
                pl.BlockSpec((D, 2 * D), lambda i: (0, 0)),
                pl.BlockSpec((D, D), lambda i: (0, 0)),
                pl.BlockSpec((1, D), lambda i: (0, 0)),
                pl.BlockSpec((1, D), lambda i: (0, 0))],
      out_specs=[pl.BlockSpec((BM, D), lambda i: (i, 0))] * 2,
      out_shape=[jax.ShapeDtypeStruct((N, D), _f32)] * 2,
  )(h, Wab, WpC, be2d, bpre2d)


def _post(h, xa, s1, s2, mn, mx, cnt, side, Wagg, Wx, Wlin,
          bpost2d, blin2d):
  BM = 1000
  grid = (N // BM,)

  def body(h_ref, xa_ref, s1_ref, s2_ref, mn_ref, mx_ref, cnt_ref,
           side_ref, wagg, wx, wlin, bp_r, bl_r, o_ref):
    i = pl.program_id(0)
    s1v = s1_ref[...]
    s2v = s2_ref[...]
    mnv = mn_ref[...]
    mxv = mx_ref[...]
    cnt0 = cnt_ref[...][:, 0:1]
    side_v = side_ref[...]
    row_pos = (lax.broadcasted_iota(_i32, (BM, 1), 0)
               + i * BM).astype(_f32)
    for w in range(_NW):
      id_w = side_v[5 * w + 4:5 * w + 5, 16:17]
      mask = row_pos == id_w
      maskf = mask.astype(_f32)
      s1v = s1v + maskf * side_v[5 * w:5 * w + 1, :]
      s2v = s2v + maskf * side_v[5 * w + 1:5 * w + 2, :]
      mnv = jnp.where(mask, jnp.minimum(mnv, side_v[5 * w + 2:5 * w + 3, :]), mnv)
      mxv = jnp.where(mask, jnp.maximum(mxv, side_v[5 * w + 3:5 * w + 4, :]), mxv)
      cnt0 = cnt0 + maskf * side_v[5 * w + 4:5 * w + 5, 0:1]
    has = cnt0 > 0.0
    deg = jnp.maximum(cnt0, 1.0)
    inv = 1.0 / deg
    s1m = s1v * inv
    var = s2v * inv - s1m * s1m
    std = jnp.sqrt(jnp.maximum(var, 0.0) + 1e-5)
    xav = xa_ref[...]
    mean = jnp.where(has, xav + s1m, 0.0)
    mno = jnp.where(has, xav + mnv, 0.0)
    mxo = jnp.where(has, xav + mxv, 0.0)
    agg = jnp.concatenate([mean, mno, mxo, std], axis=1)
    lg = jnp.log(deg + 1.0)
    amp = lg * (1.0 / AVG_LOG)
    att = AVG_LOG / lg
    g = jnp.dot(agg, wagg[...], preferred_element_type=_f32)
    out = (jnp.dot(h_ref[...], wx[...], preferred_element_type=_f32)
           + g[:, 0:D] + amp * g[:, D:2 * D] + att * g[:, 2 * D:3 * D]
           + bp_r[...])
    o_ref[...] = jnp.dot(out, wlin[...], preferred_element_type=_f32) + bl_r[...]

  nspec = pl.BlockSpec((BM, D), lambda i: (i, 0))
  return pl.pallas_call(
      body,
      grid=grid,
      in_specs=[nspec, nspec, nspec, nspec, nspec, nspec, nspec,
                pl.BlockSpec((_NW * 5, D), lambda i: (0, 0)),
                pl.BlockSpec((4 * D, 3 * D), lambda i: (0, 0)),
                pl.BlockSpec((D, D), lambda i: (0, 0)),
                pl.BlockSpec((D, D), lambda i: (0, 0)),
                pl.BlockSpec((1, D), lambda i: (0, 0)),
                pl.BlockSpec((1, D), lambda i: (0, 0))],
      out_specs=nspec,
      out_shape=jax.ShapeDtypeStruct((N, D), _f32),
  )(h, xa, s1, s2, mn, mx, cnt, side, Wagg, Wx, Wlin, bpost2d, blin2d)


def _pool_mlp(h1, h2, h3, bt8, Wl1, bl12d, Wl2p, bl2p):
  BM = 1000
  nb = N // BM
  grid = (nb,)

  def body(h1_ref, h2_ref, h3_ref, bt_ref, w1, b1, w2, b2, o_ref, acc):
    i = pl.program_id(0)

    @pl.when(i == 0)
    def _():
      acc[...] = jnp.zeros_like(acc)

    col = bt_ref[...]
    oh = (col == lax.broadcasted_iota(_i32, (1, B), 1)).astype(_f32)
    hcat = jnp.concatenate([h1_ref[...], h2_ref[...], h3_ref[...]], axis=1)
    acc[...] += lax.dot_general(oh, hcat, (((0,), (0,)), ((), ())),
                                preferred_element_type=_f32)

    @pl.when(i == nb - 1)
    def _():
      p = acc[...]
      hmid = jnp.maximum(jnp.dot(p, w1[...], preferred_element_type=_f32)
                         + b1[...], 0.0)
      o_ref[...] = jnp.dot(hmid, w2[...], preferred_element_type=_f32) + b2[...]

  nspec = pl.BlockSpec((BM, D), lambda i: (i, 0))
  return pl.pallas_call(
      body,
      grid=grid,
      in_specs=[nspec, nspec, nspec,
                pl.BlockSpec((BM, 1), lambda i: (i, 0)),
                pl.BlockSpec((3 * D, 3 * D), lambda i: (0, 0)),
                pl.BlockSpec((1, 3 * D), lambda i: (0, 0)),
                pl.BlockSpec((3 * D, D), lambda i: (0, 0)),
                pl.BlockSpec((1, D), lambda i: (0, 0))],
      out_specs=pl.BlockSpec((B, D), lambda i: (0, 0)),
      out_shape=jax.ShapeDtypeStruct((B, D), _f32),
      scratch_shapes=[pltpu.VMEM((B, 3 * D), _f32)],
  )(h1, h2, h3, bt8, Wl1, bl12d, Wl2p, bl2p)


def _sc_edge_stats(xb, ec, src_s, perm, dst_s, fwrep, gapfrom, meta,
                   n=N, e=E, interpret=False):
  mesh = plsc.VectorSubcoreMesh(core_axis_name="c", subcore_axis_name="s")
  outs = tuple([jax.ShapeDtypeStruct((n, D), _f32)] * 5
               + [jax.ShapeDtypeStruct((_NW * 5, D), _f32)])

  cp = pltpu.CompilerParams()
  if "needs_layout_passes" in pltpu.CompilerParams.__dataclass_fields__:
    cp = dataclasses.replace(cp, needs_layout_passes=False)

  @functools.partial(
      pl.kernel, out_type=outs, mesh=mesh, compiler_params=cp,
      interpret=interpret,
      scratch_types=[
          pltpu.VMEM((_NW * 16,), _i32),
          pltpu.VMEM((_C,), _i32),
          pltpu.VMEM((_C,), _i32),
          pltpu.VMEM((_C,), _i32),
          pltpu.VMEM((_C,), _i32),
          pltpu.VMEM((_C,), _i32),
          pltpu.VMEM((_C, D), _f32),
          pltpu.VMEM((_C, D), _f32),
          pltpu.VMEM((5, D), _f32),
          pltpu.VMEM((5, D), _f32),
          pltpu.SemaphoreType.DMA,
          pltpu.SemaphoreType.DMA,
          pltpu.SemaphoreType.DMA,
      ])
  def kern(xb_h, ec_h, src_h, perm_h, dst_h, fw_h, gap_h, meta_h,
           s1_h, s2_h, mn_h, mx_h, cnt_h, side_h,
           mbuf, sbuf, pbuf, dbuf, fbuf, gbuf, xbuf, ebuf, stg, zrow,
           csem, gsem, fsem):
    w = lax.axis_index("s") * _NC + lax.axis_index("c")
    pltpu.sync_copy(meta_h, mbuf)
    z16 = jnp.zeros((16,), _f32)
    for i in range(5):
      for j in range(8):
        zrow[i, pl.ds(j * 16, 16)] = z16

    rec = mbuf[pl.ds(w * 16, 16)]
    lo = rec[0]
    hi = rec[1]
    tailgap = rec[2]
    k0 = lax.div(lo, _C)
    k1 = lax.div(hi + _C - 1, _C)

    def gap_row(r, carry):
      pltpu.sync_copy(zrow.at[0], s1_h.at[r])
      pltpu.sync_copy(zrow.at[1], s2_h.at[r])
      pltpu.sync_copy(zrow.at[2], mn_h.at[r])
      pltpu.sync_copy(zrow.at[3], mx_h.at[r])
      pltpu.sync_copy(zrow.at[4], cnt_h.at[r])
      return carry

    def drain():
      pltpu.make_async_copy(stg.at[0], s1_h.at[0], fsem).wait()
      pltpu.make_async_copy(stg.at[1], s2_h.at[0], fsem).wait()
      pltpu.make_async_copy(stg.at[2], mn_h.at[0], fsem).wait()
      pltpu.make_async_copy(stg.at[3], mx_h.at[0], fsem).wait()
      pltpu.make_async_copy(stg.at[4], cnt_h.at[0], fsem).wait()

    inf16 = jnp.full((16,), jnp.inf, _f32)
    ninf16 = jnp.full((16,), -jnp.inf, _f32)

    def chunk_body(kk, carry):
      kbase = kk * _C
      h1 = pltpu.async_copy(src_h.at[pl.ds(kbase, _C)], sbuf, csem)
      h2 = pltpu.async_copy(perm_h.at[pl.ds(kbase, _C)], pbuf, csem)
      h3 = pltpu.async_copy(dst_h.at[pl.ds(kbase, _C)], dbuf, csem)
      h4 = pltpu.async_copy(fw_h.at[pl.ds(kbase, _C)], fbuf, csem)
      h5 = pltpu.async_copy(gap_h.at[pl.ds(kbase, _C)], gbuf, csem)
      h1.wait()
      h2.wait()
      h3.wait()
      h4.wait()
      h5.wait()
      g1 = pltpu.async_copy(xb_h.at[sbuf], xbuf, gsem)
      g2 = pltpu.async_copy(ec_h.at[pbuf], ebuf, gsem)
      g1.wait()
      g2.wait()

      def group_body(g, gcarry):
        gb = g * 16
        word = fbuf[pl.ds(gb, 16)][0]
        carry_u = gcarry
        for u in range(16):
          segn, cntf, s1v, s2v, mnv, mxv = carry_u
          li = gb + u
          s1n, s2n, mnn, mxn = [], [], [], []
          for j in range(8):
            t = xbuf[li, pl.ds(16 * j, 16)] + ebuf[li, pl.ds(16 * j, 16)]
            s1n.append(s1v[j] + t)
            s2n.append(s2v[j] + t * t)
            mnn.append(jnp.minimum(mnv[j], t))
            mxn.append(jnp.maximum(mxv[j], t))
          cntn = cntf + 1.0
          fl = lax.shift_right_logical(word, u) & 1 if u else word & 1

          @pl.when(fl == 1)
          def _(s1n=s1n, s2n=s2n, mnn=mnn, mxn=mxn, cntn=cntn,
                gb=gb, u=u, segn=segn):
            @pl.when(segn > 0)
            def _():
              drain()
            d = dbuf[pl.ds(gb, 16)][u]
            gf = gbuf[pl.ds(gb, 16)][u]
            lax.fori_loop(gf, d, gap_row, 0)
            for j in range(8):
              stg[0, pl.ds(16 * j, 16)] = s1n[j]
              stg[1, pl.ds(16 * j, 16)] = s2n[j]
              stg[2, pl.ds(16 * j, 16)] = mnn[j]
              stg[3, pl.ds(16 * j, 16)] = mxn[j]
            cv = jnp.full((16,), cntn, _f32)
            for j in range(8):
              stg[4, pl.ds(16 * j, 16)] = cv
            pltpu.async_copy(stg.at[0], s1_h.at[d], fsem)
            pltpu.async_copy(stg.at[1], s2_h.at[d], fsem)
            pltpu.async_copy(stg.at[2], mn_h.at[d], fsem)
            pltpu.async_copy(stg.at[3], mx_h.at[d], fsem)
            pltpu.async_copy(stg.at[4], cnt_h.at[d], fsem)

          flb = fl == 1
          fmask = jnp.full((16,), fl, _i32) > 0
          s1o = tuple(jnp.where(fmask, z16, v) for v in s1n)
          s2o = tuple(jnp.where(fmask, z16, v) for v in s2n)
          mno = tuple(jnp.where(fmask, inf16, v) for v in mnn)
          mxo = tuple(jnp.where(fmask, ninf16, v) for v in mxn)
          carry_u = (segn + fl, jnp.where(flb, 0.0, cntn),
                     s1o, s2o, mno, mxo)
        return carry_u

      g0 = lax.div(jnp.maximum(lo, kbase) - kbase, 16)
      g1i = lax.div(jnp.minimum(hi, kbase + _C) - kbase, 16)
      return lax.fori_loop(g0, g1i, group_body, carry)

    init = (jnp.int32(0), jnp.float32(0.0),
            tuple([z16] * 8), tuple([z16] * 8),
            tuple([inf16] * 8), tuple([ninf16] * 8))
    fin = lax.fori_loop(k0, k1, chunk_body, init)
    segf, cntff, s1f, s2f, mnf, mxf = fin

    @pl.when(segf > 0)
    def _():
      drain()

    taild = rec[3]
    lax.fori_loop(tailgap, taild, gap_row, 0)

    @pl.when(w == _NW - 1)
    def _():
      lax.fori_loop(taild + 1, n, gap_row, 0)

    for j in range(8):
      stg[0, pl.ds(16 * j, 16)] = s1f[j]
      stg[1, pl.ds(16 * j, 16)] = s2f[j]
      stg[2, pl.ds(16 * j, 16)] = mnf[j]
      stg[3, pl.ds(16 * j, 16)] = mxf[j]
    stg[4, pl.ds(0, 16)] = jnp.full((16,), cntff, _f32)
    stg[4, pl.ds(16, 16)] = jnp.full((16,), taild.astype(_f32), _f32)
    pltpu.async_copy(stg.at[0], side_h.at[w * 5 + 0], fsem)
    pltpu.async_copy(stg.at[1], side_h.at[w * 5 + 1], fsem)
    pltpu.async_copy(stg.at[2], side_h.at[w * 5 + 2], fsem)
    pltpu.async_copy(stg.at[3], side_h.at[w * 5 + 3], fsem)
    pltpu.async_copy(stg.at[4], side_h.at[w * 5 + 4], fsem)
    pltpu.make_async_copy(stg.at[0], side_h.at[w * 5 + 0], fsem).wait()
    pltpu.make_async_copy(stg.at[1], side_h.at[w * 5 + 1], fsem).wait()
    pltpu.make_async_copy(stg.at[2], side_h.at[w * 5 + 2], fsem).wait()
    pltpu.make_async_copy(stg.at[3], side_h.at[w * 5 + 3], fsem).wait()
    pltpu.make_async_copy(stg.at[4], side_h.at[w * 5 + 4], fsem).wait()

  return kern(xb, ec, src_s, perm, dst_s, fwrep, gapfrom, meta)


def kernel(x, edge_index, edge_attr, batch,
           We1, be1, Wpre1, bpre1, Wpost1, bpost1, Wlin1, blin1,
           We2, be2, Wpre2, bpre2, Wpost2, bpost2, Wlin2, blin2,
           We3, be3, Wpre3, bpre3, Wpost3, bpost3, Wlin3, blin3,
           Wl1, bl1, Wl2, bl2):
  src = edge_index[0].astype(_i32)
  dst = edge_index[1].astype(_i32)

  perm = jnp.argsort(dst).astype(_i32)
  dst_s = jnp.take(dst, perm)
  src_s = jnp.take(src, perm)
  neq = (dst_s[1:] != dst_s[:-1]).astype(_i32)
  flags = jnp.concatenate([neq, jnp.ones((1,), _i32)])
  is_start = jnp.concatenate([jnp.ones((1,), _i32), neq])
  prevd = jnp.where(is_start == 1,
                    jnp.concatenate([jnp.full((1,), -1, _i32), dst_s[:-1]]),
                    -1)
  gapfrom = lax.cummax(prevd) + 1
  fw = (flags.reshape(E // 16, 16)
        << jnp.arange(16, dtype=_i32)[None, :]).sum(axis=1, dtype=_i32)
  fwrep = jnp.repeat(fw, 16)
  los = ((jnp.arange(_NW + 1, dtype=_i32) * (E // _NW)) // 16 * 16)
  taild = jnp.take(dst_s, los[1:] - 1)
  tailgap = jnp.take(gapfrom, los[1:] - 1)
  zc = jnp.zeros((_NW,), _i32)
  meta = jnp.stack([los[:-1], los[1:], tailgap, taild]
                   + [zc] * 12, axis=1).reshape(-1)

  bt8 = batch.astype(_i32)[:, None]

  layers = []
  for (We, be, Wpre, bpre, Wpost, bpost, Wlin, blin) in (
      (We1, be1, Wpre1, bpre1, Wpost1, bpost1, Wlin1, blin1),
      (We2, be2, Wpre2, bpre2, Wpost2, bpost2, Wlin2, blin2),
      (We3, be3, Wpre3, bpre3, Wpost3, bpost3, Wlin3, blin3)):
    Wab = jnp.concatenate([Wpre[:D], Wpre[D:2 * D]], axis=1)
    WpC = Wpre[2 * D:]
    Wx = Wpost[:D]
    Wagg = jnp.concatenate([Wpost[D:D + 4 * D],
                            Wpost[D + 4 * D:D + 8 * D],
                            Wpost[D + 8 * D:D + 12 * D]], axis=1)
    layers.append(dict(We=We, Wab=Wab, WpC=WpC, be2d=be[None, :],
                       bpre2d=bpre[None, :], Wx=Wx, Wagg=Wagg,
                       Wlin=Wlin, bpost2d=bpost[None, :],
                       blin2d=blin[None, :]))

  ecs = _ec_matmuls(edge_attr,
                    layers[0]["We"], layers[0]["WpC"],
                    layers[1]["We"], layers[1]["WpC"],
                    layers[2]["We"], layers[2]["WpC"])

  h = x
  hs = []
  for l in range(3):
    ly = layers[l]
    xa, xb = _node_pre(h, ly["Wab"], ly["WpC"], ly["be2d"], ly["bpre2d"])
    s1, s2, mn, mx, cnt, side = _sc_edge_stats(
        xb, ecs[l], src_s, perm, dst_s, fwrep, gapfrom, meta)
    h = _post(h, xa, s1, s2, mn, mx, cnt, side, ly["Wagg"], ly["Wx"],
              ly["Wlin"], ly["bpost2d"], ly["blin2d"])
    hs.append(h)

  Wl2p = jnp.pad(Wl2, ((0, 0), (0, D - 1)))
  bl2p = jnp.pad(bl2[None, :], ((0, 0), (0, D - 1)))
  y = _pool_mlp(hs[0], hs[1], hs[2], bt8, Wl1, bl1[None, :], Wl2p, bl2p)
  return y[:, 0:1]


# speedup vs baseline: 1.0282x; 1.0282x over previous
import dataclasses
import functools

import jax
import jax.numpy as jnp
import numpy as np
from jax import lax
from jax.experimental import pallas as pl
from jax.experimental.pallas import tpu as pltpu
from jax.experimental.pallas import tpu_sc as plsc

N = 10000
E = 320000
D = 128
ED = 16
B = 64

_DEG_HIST = np.array([0] * 32 + [10000], dtype=np.float64)
_bins_tmp = np.arange(len(_DEG_HIST))
AVG_LOG = float((np.log(_bins_tmp + 1.0) * _DEG_HIST).sum() / _DEG_HIST.sum())

_NC = 2
_NS = 16
_NW = _NC * _NS
_C = 256

_f32 = jnp.float32
_i32 = jnp.int32


def _ec_matmuls(ea, We1, Wp1, We2, Wp2, We3, Wp3):
  BM = 2000
  grid = (E // BM,)

  def body(a_ref, we1, wp1, we2, wp2, we3, wp3, o1, o2, o3):
    a = a_ref[...]
    for we, wp, o in ((we1, wp1, o1), (we2, wp2, o2), (we3, wp3, o3)):
      w = jnp.dot(we[...], wp[...], preferred_element_type=_f32)
      o[...] = jnp.dot(a, w, preferred_element_type=_f32)

  wspec = pl.BlockSpec((ED, D), lambda i: (0, 0))
  pspec = pl.BlockSpec((D, D), lambda i: (0, 0))
  return pl.pallas_call(
      body,
      grid=grid,
      in_specs=[pl.BlockSpec((BM, ED), lambda i: (i, 0)),
                wspec, pspec, wspec, pspec, wspec, pspec],
      out_specs=[pl.BlockSpec((BM, D), lambda i: (i, 0))] * 3,
      out_shape=[jax.ShapeDtypeStruct((E, D), _f32)] * 3,
  )(ea, We1, Wp1, We2, Wp2, We3, Wp3)


def _node_pre(h, Wab, WpC, be2d, bpre2d):
  BM = 1000
  grid = (N // BM,)

  def body(h_ref, wab, wpc, be_r, bp_r, o1, o2):
    c0 = jnp.dot(be_r[...], wpc[...], preferred_element_type=_f32) + bp_r[...]
    r = jnp.dot(h_ref[...], wab[...], preferred_element_type=_f32)
    o1[...] = r[:, 0:D] + c0
    o2[...] = r[:, D:2 * D]

  return pl.pallas_call(
      body,
      grid=grid,
      in_specs=[pl.BlockSpec((BM, D), lambda i: (i, 0)),
                pl.BlockSpec((D, 2 * D), lambda i: (0, 0)),
                pl.BlockSpec((D, D), lambda i: (0, 0)),
                pl.BlockSpec((1, D), lambda i: (0, 0)),
                pl.BlockSpec((1, D), lambda i: (0, 0))],
      out_specs=[pl.BlockSpec((BM, D), lambda i: (i, 0))] * 2,
      out_shape=[jax.ShapeDtypeStruct((N, D), _f32)] * 2,
  )(h, Wab, WpC, be2d, bpre2d)


def _post(h, xa, s1, s2, mn, mx, cnt, side, Wagg, Wx, Wlin,
          bpost2d, blin2d):
  BM = 1000
  grid = (N // BM,)

  def body(h_ref, xa_ref, s1_ref, s2_ref, mn_ref, mx_ref, cnt_ref,
           side_ref, wagg, wx, wlin, bp_r, bl_r, o_ref):
    i = pl.program_id(0)
    s1v = s1_ref[...]
    s2v = s2_ref[...]
    mnv = mn_ref[...]
    mxv = mx_ref[...]
    cnt0 = cnt_ref[...][:, 0:1]
    side_v = side_ref[...]
    row_pos = (lax.broadcasted_iota(_i32, (BM, 1), 0)
               + i * BM).astype(_f32)
    for w in range(_NW):
      id_w = side_v[5 * w + 4:5 * w + 5, 16:17]
      mask = row_pos == id_w
      maskf = mask.astype(_f32)
      s1v = s1v + maskf * side_v[5 * w:5 * w + 1, :]
      s2v = s2v + maskf * side_v[5 * w + 1:5 * w + 2, :]
      mnv = jnp.where(mask, jnp.minimum(mnv, side_v[5 * w + 2:5 * w + 3, :]), mnv)
      mxv = jnp.where(mask, jnp.maximum(mxv, side_v[5 * w + 3:5 * w + 4, :]), mxv)
      cnt0 = cnt0 + maskf * side_v[5 * w + 4:5 * w + 5, 0:1]
    has = cnt0 > 0.0
    deg = jnp.maximum(cnt0, 1.0)
    inv = 1.0 / deg
    s1m = s1v * inv
    var = s2v * inv - s1m * s1m
    std = jnp.sqrt(jnp.maximum(var, 0.0) + 1e-5)
    xav = xa_ref[...]
    mean = jnp.where(has, xav + s1m, 0.0)
    mno = jnp.where(has, xav + mnv, 0.0)
    mxo = jnp.where(has, xav + mxv, 0.0)
    agg = jnp.concatenate([mean, mno, mxo, std], axis=1)
    lg = jnp.log(deg + 1.0)
    amp = lg * (1.0 / AVG_LOG)
    att = AVG_LOG / lg
    g = jnp.dot(agg, wagg[...], preferred_element_type=_f32)
    out = (jnp.dot(h_ref[...], wx[...], preferred_element_type=_f32)
           + g[:, 0:D] + amp * g[:, D:2 * D] + att * g[:, 2 * D:3 * D]
           + bp_r[...])
    o_ref[...] = jnp.dot(out, wlin[...], preferred_element_type=_f32) + bl_r[...]

  nspec = pl.BlockSpec((BM, D), lambda i: (i, 0))
  return pl.pallas_call(
      body,
      grid=grid,
      in_specs=[nspec, nspec, nspec, nspec, nspec, nspec, nspec,
                pl.BlockSpec((_NW * 5, D), lambda i: (0, 0)),
                pl.BlockSpec((4 * D, 3 * D), lambda i: (0, 0)),
                pl.BlockSpec((D, D), lambda i: (0, 0)),
                pl.BlockSpec((D, D), lambda i: (0, 0)),
                pl.BlockSpec((1, D), lambda i: (0, 0)),
                pl.BlockSpec((1, D), lambda i: (0, 0))],
      out_specs=nspec,
      out_shape=jax.ShapeDtypeStruct((N, D), _f32),
  )(h, xa, s1, s2, mn, mx, cnt, side, Wagg, Wx, Wlin, bpost2d, blin2d)


def _pool_mlp(h1, h2, h3, bt8, Wl1, bl12d, Wl2p, bl2p):
  BM = 1000
  nb = N // BM
  grid = (nb,)

  def body(h1_ref, h2_ref, h3_ref, bt_ref, w1, b1, w2, b2, o_ref, acc):
    i = pl.program_id(0)

    @pl.when(i == 0)
    def _():
      acc[...] = jnp.zeros_like(acc)

    col = bt_ref[...]
    oh = (col == lax.broadcasted_iota(_i32, (1, B), 1)).astype(_f32)
    hcat = jnp.concatenate([h1_ref[...], h2_ref[...], h3_ref[...]], axis=1)
    acc[...] += lax.dot_general(oh, hcat, (((0,), (0,)), ((), ())),
                                preferred_element_type=_f32)

    @pl.when(i == nb - 1)
    def _():
      p = acc[...]
      hmid = jnp.maximum(jnp.dot(p, w1[...], preferred_element_type=_f32)
                         + b1[...], 0.0)
      o_ref[...] = jnp.dot(hmid, w2[...], preferred_element_type=_f32) + b2[...]

  nspec = pl.BlockSpec((BM, D), lambda i: (i, 0))
  return pl.pallas_call(
      body,
      grid=grid,
      in_specs=[nspec, nspec, nspec,
                pl.BlockSpec((BM, 1), lambda i: (i, 0)),
                pl.BlockSpec((3 * D, 3 * D), lambda i: (0, 0)),
                pl.BlockSpec((1, 3 * D), lambda i: (0, 0)),
                pl.BlockSpec((3 * D, D), lambda i: (0, 0)),
                pl.BlockSpec((1, D), lambda i: (0, 0))],
      out_specs=pl.BlockSpec((B, D), lambda i: (0, 0)),
      out_shape=jax.ShapeDtypeStruct((B, D), _f32),
      scratch_shapes=[pltpu.VMEM((B, 3 * D), _f32)],
  )(h1, h2, h3, bt8, Wl1, bl12d, Wl2p, bl2p)


def _sc_edge_stats(xb, ec, src_s, perm, dst_s, fwrep, gapfrom, meta,
                   n=N, e=E, interpret=False):
  mesh = plsc.VectorSubcoreMesh(core_axis_name="c", subcore_axis_name="s")
  outs = tuple([jax.ShapeDtypeStruct((n, D), _f32)] * 5
               + [jax.ShapeDtypeStruct((_NW * 5, D), _f32)])

  cp = pltpu.CompilerParams()
  if "needs_layout_passes" in pltpu.CompilerParams.__dataclass_fields__:
    cp = dataclasses.replace(cp, needs_layout_passes=False)

  @functools.partial(
      pl.kernel, out_type=outs, mesh=mesh, compiler_params=cp,
      interpret=interpret,
      scratch_types=[
          pltpu.VMEM((_NW * 16,), _i32),
          pltpu.VMEM((_C,), _i32),
          pltpu.VMEM((_C,), _i32),
          pltpu.VMEM((_C,), _i32),
          pltpu.VMEM((_C,), _i32),
          pltpu.VMEM((_C,), _i32),
          pltpu.VMEM((_C, D), _f32),
          pltpu.VMEM((_C, D), _f32),
          pltpu.VMEM((5, D), _f32),
          pltpu.VMEM((5, D), _f32),
          pltpu.SemaphoreType.DMA,
          pltpu.SemaphoreType.DMA,
          pltpu.SemaphoreType.DMA,
      ])
  def kern(xb_h, ec_h, src_h, perm_h, dst_h, fw_h, gap_h, meta_h,
           s1_h, s2_h, mn_h, mx_h, cnt_h, side_h,
           mbuf, sbuf, pbuf, dbuf, fbuf, gbuf, xbuf, ebuf, stg, zrow,
           csem, gsem, fsem):
    w = lax.axis_index("s") * _NC + lax.axis_index("c")
    pltpu.sync_copy(meta_h, mbuf)
    z16 = jnp.zeros((16,), _f32)
    for i in range(5):
      for j in range(8):
        zrow[i, pl.ds(j * 16, 16)] = z16

    rec = mbuf[pl.ds(w * 16, 16)]
    lo = rec[0]
    hi = rec[1]
    tailgap = rec[2]
    k0 = lax.div(lo, _C)
    k1 = lax.div(hi + _C - 1, _C)

    def gap_row(r, carry):
      pltpu.sync_copy(zrow.at[0], s1_h.at[r])
      pltpu.sync_copy(zrow.at[1], s2_h.at[r])
      pltpu.sync_copy(zrow.at[2], mn_h.at[r])
      pltpu.sync_copy(zrow.at[3], mx_h.at[r])
      pltpu.sync_copy(zrow.at[4], cnt_h.at[r])
      return carry

    def drain():
      pltpu.make_async_copy(stg.at[0], s1_h.at[0], fsem).wait()
      pltpu.make_async_copy(stg.at[1], s2_h.at[0], fsem).wait()
      pltpu.make_async_copy(stg.at[2], mn_h.at[0], fsem).wait()
      pltpu.make_async_copy(stg.at[3], mx_h.at[0], fsem).wait()
      pltpu.make_async_copy(stg.at[4], cnt_h.at[0], fsem).wait()

    inf16 = jnp.full((16,), jnp.inf, _f32)
    ninf16 = jnp.full((16,), -jnp.inf, _f32)

    def chunk_body(kk, carry):
      kbase = kk * _C
      h1 = pltpu.async_copy(src_h.at[pl.ds(kbase, _C)], sbuf, csem)
      h2 = pltpu.async_copy(perm_h.at[pl.ds(kbase, _C)], pbuf, csem)
      h3 = pltpu.async_copy(dst_h.at[pl.ds(kbase, _C)], dbuf, csem)
      h4 = pltpu.async_copy(fw_h.at[pl.ds(kbase, _C)], fbuf, csem)
      h5 = pltpu.async_copy(gap_h.at[pl.ds(kbase, _C)], gbuf, csem)
      h1.wait()
      h2.wait()
      h3.wait()
      h4.wait()
      h5.wait()
      g1 = pltpu.async_copy(xb_h.at[sbuf], xbuf, gsem)
      g2 = pltpu.async_copy(ec_h.at[pbuf], ebuf, gsem)
      g1.wait()
      g2.wait()

      def group_body(g, gcarry):
        gb = g * 16
        word = fbuf[pl.ds(gb, 16)][0]
        carry_u = gcarry
        for u in range(16):
          segn, cntf, s1v, s2v, mnv, mxv = carry_u
          li = gb + u
          s1n, s2n, mnn, mxn = [], [], [], []
          for j in range(8):
            t = xbuf[li, pl.ds(16 * j, 16)] + ebuf[li, pl.ds(16 * j, 16)]
            s1n.append(s1v[j] + t)
            s2n.append(s2v[j] + t * t)
            mnn.append(jnp.minimum(mnv[j], t))
            mxn.append(jnp.maximum(mxv[j], t))
          cntn = cntf + 1.0
          fl = lax.shift_right_logical(word, u) & 1 if u else word & 1

          @pl.when(fl == 1)
          def _(s1n=s1n, s2n=s2n, mnn=mnn, mxn=mxn, cntn=cntn,
                gb=gb, u=u, segn=segn):
            @pl.when(segn > 0)
            def _():
              drain()
            d = dbuf[pl.ds(gb, 16)][u]
            gf = gbuf[pl.ds(gb, 16)][u]
            lax.fori_loop(gf, d, gap_row, 0)
            for j in range(8):
              stg[0, pl.ds(16 * j, 16)] = s1n[j]
              stg[1, pl.ds(16 * j, 16)] = s2n[j]
              stg[2, pl.ds(16 * j, 16)] = mnn[j]
              stg[3, pl.ds(16 * j, 16)] = mxn[j]
            cv = jnp.full((16,), cntn, _f32)
            for j in range(8):
              stg[4, pl.ds(16 * j, 16)] = cv
            pltpu.async_copy(stg.at[0], s1_h.at[d], fsem)
            pltpu.async_copy(stg.at[1], s2_h.at[d], fsem)
            pltpu.async_copy(stg.at[2], mn_h.at[d], fsem)
            pltpu.async_copy(stg.at[3], mx_h.at[d], fsem)
            pltpu.async_copy(stg.at[4], cnt_h.at[d], fsem)

          flb = fl == 1
          fmask = jnp.full((16,), fl, _i32) > 0
          s1o = tuple(jnp.where(fmask, z16, v) for v in s1n)
          s2o = tuple(jnp.where(fmask, z16, v) for v in s2n)
          mno = tuple(jnp.where(fmask, inf16, v) for v in mnn)
          mxo = tuple(jnp.where(fmask, ninf16, v) for v in mxn)
          carry_u = (segn + fl, jnp.where(flb, 0.0, cntn),
                     s1o, s2o, mno, mxo)
        return carry_u

      g0 = lax.div(jnp.maximum(lo, kbase) - kbase, 16)
      g1i = lax.div(jnp.minimum(hi, kbase + _C) - kbase, 16)
      return lax.fori_loop(g0, g1i, group_body, carry)

    init = (jnp.int32(0), jnp.float32(0.0),
            tuple([z16] * 8), tuple([z16] * 8),
            tuple([inf16] * 8), tuple([ninf16] * 8))
    fin = lax.fori_loop(k0, k1, chunk_body, init)
    segf, cntff, s1f, s2f, mnf, mxf = fin

    @pl.when(segf > 0)
    def _():
      drain()

    taild = rec[3]
    lax.fori_loop(tailgap, taild, gap_row, 0)

    @pl.when(w == _NW - 1)
    def _():
      lax.fori_loop(taild + 1, n, gap_row, 0)

    for j in range(8):
      stg[0, pl.ds(16 * j, 16)] = s1f[j]
      stg[1, pl.ds(16 * j, 16)] = s2f[j]
      stg[2, pl.ds(16 * j, 16)] = mnf[j]
      stg[3, pl.ds(16 * j, 16)] = mxf[j]
    stg[4, pl.ds(0, 16)] = jnp.full((16,), cntff, _f32)
    stg[4, pl.ds(16, 16)] = jnp.full((16,), taild.astype(_f32), _f32)
    pltpu.async_copy(stg.at[0], side_h.at[w * 5 + 0], fsem)
    pltpu.async_copy(stg.at[1], side_h.at[w * 5 + 1], fsem)
    pltpu.async_copy(stg.at[2], side_h.at[w * 5 + 2], fsem)
    pltpu.async_copy(stg.at[3], side_h.at[w * 5 + 3], fsem)
    pltpu.async_copy(stg.at[4], side_h.at[w * 5 + 4], fsem)
    pltpu.make_async_copy(stg.at[0], side_h.at[w * 5 + 0], fsem).wait()
    pltpu.make_async_copy(stg.at[1], side_h.at[w * 5 + 1], fsem).wait()
    pltpu.make_async_copy(stg.at[2], side_h.at[w * 5 + 2], fsem).wait()
    pltpu.make_async_copy(stg.at[3], side_h.at[w * 5 + 3], fsem).wait()
    pltpu.make_async_copy(stg.at[4], side_h.at[w * 5 + 4], fsem).wait()

  return kern(xb, ec, src_s, perm, dst_s, fwrep, gapfrom, meta)


def kernel(x, edge_index, edge_attr, batch,
           We1, be1, Wpre1, bpre1, Wpost1, bpost1, Wlin1, blin1,
           We2, be2, Wpre2, bpre2, Wpost2, bpost2, Wlin2, blin2,
           We3, be3, Wpre3, bpre3, Wpost3, bpost3, Wlin3, blin3,
           Wl1, bl1, Wl2, bl2):
  src = edge_index[0].astype(_i32)
  dst = edge_index[1].astype(_i32)

  perm = jnp.argsort(dst).astype(_i32)
  dst_s = jnp.take(dst, perm)
  src_s = jnp.take(src, perm)
  neq = (dst_s[1:] != dst_s[:-1]).astype(_i32)
  flags = jnp.concatenate([neq, jnp.ones((1,), _i32)])
  is_start = jnp.concatenate([jnp.ones((1,), _i32), neq])
  prevd = jnp.where(is_start == 1,
                    jnp.concatenate([jnp.full((1,), -1, _i32), dst_s[:-1]]),
                    -1)
  gapfrom = lax.cummax(prevd) + 1
  fw = (flags.reshape(E // 16, 16)
        << jnp.arange(16, dtype=_i32)[None, :]).sum(axis=1, dtype=_i32)
  fwrep = jnp.repeat(fw, 16)
  los = ((jnp.arange(_NW + 1, dtype=_i32) * (E // _NW)) // 16 * 16)
  taild = jnp.take(dst_s, los[1:] - 1)
  tailgap = jnp.take(gapfrom, los[1:] - 1)
  zc = jnp.zeros((_NW,), _i32)
  meta = jnp.stack([los[:-1], los[1:], tailgap, taild]
                   + [zc] * 12, axis=1).reshape(-1)

  bt8 = batch.astype(_i32)[:, None]

  layers = []
  for (We, be, Wpre, bpre, Wpost, bpost, Wlin, blin) in (
      (We1, be1, Wpre1, bpre1, Wpost1, bpost1, Wlin1, blin1),
      (We2, be2, Wpre2, bpre2, Wpost2, bpost2, Wlin2, blin2),
      (We3, be3, Wpre3, bpre3, Wpost3, bpost3, Wlin3, blin3)):
    Wab = jnp.concatenate([Wpre[:D], Wpre[D:2 * D]], axis=1)
    WpC = Wpre[2 * D:]
    Wx = Wpost[:D]
    Wagg = jnp.concatenate([Wpost[D:D + 4 * D],
                            Wpost[D + 4 * D:D + 8 * D],
                            Wpost[D + 8 * D:D + 12 * D]], axis=1)
    layers.append(dict(We=We, Wab=Wab, WpC=WpC, be2d=be[None, :],
                       bpre2d=bpre[None, :], Wx=Wx, Wagg=Wagg,
                       Wlin=Wlin, bpost2d=bpost[None, :],
                       blin2d=blin[None, :]))

  ecs = _ec_matmuls(edge_attr,
                    layers[0]["We"], layers[0]["WpC"],
                    layers[1]["We"], layers[1]["WpC"],
                    layers[2]["We"], layers[2]["WpC"])

  h = x
  hs = []
  for l in range(3):
    ly = layers[l]
    xa, xb = _node_pre(h, ly["Wab"], ly["WpC"], ly["be2d"], ly["bpre2d"])
    s1, s2, mn, mx, cnt, side = _sc_edge_stats(
        xb, ecs[l], src_s, perm, dst_s, fwrep, gapfrom, meta)
    h = _post(h, xa, s1, s2, mn, mx, cnt, side, ly["Wagg"], ly["Wx"],
              ly["Wlin"], ly["bpost2d"], ly["blin2d"])
    hs.append(h)

  Wl2p = jnp.pad(Wl2, ((0, 0), (0, D - 1)))
  bl2p = jnp.pad(bl2[None, :], ((0, 0), (0, D - 1)))
  y = _pool_mlp(hs[0], hs[1], hs[2], bt8, Wl1, bl1[None, :], Wl2p, bl2p)
  return y[:, 0:1]
